# Initial kernel scaffold; baseline (speedup 1.0000x reference)
import jax
import jax.numpy as jnp
from jax.experimental import pallas as pl


def kernel(x, edge_index, W1, b1, W2, b2):
    raise NotImplementedError("write your pallas kernel here")


# speedup vs baseline: 14.7675x; 14.7675x over previous
import functools

import jax
import jax.numpy as jnp
from jax import lax
from jax.experimental import pallas as pl
from jax.experimental.pallas import tpu as pltpu
from jax.experimental.pallas import tpu_sc as plsc

N = 10000
E = 320000
D = 128
DH = D // 2

NC = 2
NS = 16
NW = NC * NS
EPT = E // NS
CH = 80
NCHUNK = EPT // CH
ZR = 80
NZCH = N // ZR
EPW = E // NW

_mesh = plsc.VectorSubcoreMesh(
    core_axis_name="c", subcore_axis_name="s", num_cores=NC, num_subcores=NS
)


@functools.partial(
    pl.kernel,
    out_type=jax.ShapeDtypeStruct((NW, N), jnp.float32),
    mesh=_mesh,
    scratch_types=[
        pltpu.VMEM((EPW,), jnp.int32),
        pltpu.VMEM((N,), jnp.float32),
    ],
    compiler_params=pltpu.CompilerParams(needs_layout_passes=False),
)
def _degree_kernel(dst_hbm, hist_hbm, dstv, hist):
    c = lax.axis_index("c")
    s = lax.axis_index("s")
    wid = c * NS + s
    pltpu.sync_copy(dst_hbm.at[wid], dstv)

    zeros16 = jnp.zeros((16,), jnp.float32)
    ones16 = jnp.ones((16,), jnp.float32)

    def zero_body(i, carry):
        hist[pl.ds(i * 16, 16)] = zeros16
        return carry

    lax.fori_loop(0, N // 16, zero_body, None)

    def edge_body(i, carry):
        idx = dstv[pl.ds(i * 16, 16)]
        plsc.addupdate_scatter(hist, [idx], ones16)
        return carry

    lax.fori_loop(0, EPW // 16, edge_body, None)
    pltpu.sync_copy(hist, hist_hbm.at[wid])


@functools.partial(
    pl.kernel,
    out_type=jax.ShapeDtypeStruct((NC, N, DH), jnp.float32),
    mesh=_mesh,
    scratch_types=[
        pltpu.VMEM((NCHUNK, CH), jnp.int32),
        pltpu.VMEM((NCHUNK, CH), jnp.int32),
        pltpu.VMEM((CH, DH), jnp.float32),
        pltpu.VMEM((ZR, DH), jnp.float32),
        pltpu.VMEM_SHARED((N, DH), jnp.float32),
        pltpu.SemaphoreType.DMA,
    ],
    compiler_params=pltpu.CompilerParams(
        needs_layout_passes=False, use_tc_tiling_on_sc=False
    ),
)
def _scatter_kernel(hs_hbm, src_hbm, dst_hbm, out_hbm, srcm, dstm, rows, zbuf,
                    accum, sem):
    c = lax.axis_index("c")
    s = lax.axis_index("s")

    pltpu.sync_copy(src_hbm.at[s], srcm)
    pltpu.sync_copy(dst_hbm.at[s], dstm)

    zeros16 = jnp.zeros((16,), jnp.float32)

    def zrow(i, carry):
        def zcol(j, inner):
            zbuf[i, pl.ds(j * 16, 16)] = zeros16
            return inner

        return lax.fori_loop(0, DH // 16, zcol, carry)

    lax.fori_loop(0, ZR, zrow, None)

    def zchunk(j, carry):
        chunk = s * 8 + j

        @pl.when(chunk < NZCH)
        def _():
            pltpu.sync_copy(zbuf, accum.at[pl.ds(chunk * ZR, ZR)])

        return carry

    lax.fori_loop(0, 8, zchunk, None)
    plsc.subcore_barrier()

    def edge_chunk(k, carry):
        pltpu.async_copy(hs_hbm.at[c].at[srcm.at[k]], rows, sem).wait()
        pltpu.sync_copy(rows, accum.at[dstm.at[k]], add=True)
        return carry

    lax.fori_loop(0, NCHUNK, edge_chunk, None)
    plsc.subcore_barrier()

    def fchunk(j, carry):
        chunk = s * 8 + j

        @pl.when(chunk < NZCH)
        def _():
            pltpu.sync_copy(accum.at[pl.ds(chunk * ZR, ZR)],
                            out_hbm.at[c, pl.ds(chunk * ZR, ZR)])

        return carry

    lax.fori_loop(0, 8, fchunk, None)


RB = 2000
GRID = N // RB


def _mm1_body(x_ref, w_ref, hist_ref, hs_ref, dinv_ref):
    deg = jnp.sum(hist_ref[...], axis=1) + 1.0
    dinv = lax.rsqrt(deg)
    h = jnp.dot(x_ref[...], w_ref[...], preferred_element_type=jnp.float32)
    hs = h * dinv[:, None]
    hs_ref[0] = hs[:, :DH]
    hs_ref[1] = hs[:, DH:]
    dinv_ref[...] = dinv[:, None]


def _mid_body(part_ref, hs1_ref, dinv_ref, b1_ref, w2_ref, hs2_ref):
    hs1 = jnp.concatenate([part_ref[0] + hs1_ref[0], part_ref[1] + hs1_ref[1]],
                          axis=1)
    dinv = dinv_ref[...]
    t = dinv * hs1 + b1_ref[...]
    t = jnp.maximum(t, 0.0)
    h2 = jnp.dot(t, w2_ref[...], preferred_element_type=jnp.float32)
    hs2 = h2 * dinv
    hs2_ref[0] = hs2[:, :DH]
    hs2_ref[1] = hs2[:, DH:]


def _out_body(part_ref, hs2_ref, dinv_ref, b2_ref, o_ref):
    agg = jnp.concatenate([part_ref[0] + hs2_ref[0], part_ref[1] + hs2_ref[1]],
                          axis=1)
    o_ref[...] = dinv_ref[...] * agg + b2_ref[...]


def _mm1(x, w1, hist):
    return pl.pallas_call(
        _mm1_body,
        grid=(GRID,),
        in_specs=[
            pl.BlockSpec((RB, D), lambda i: (i, 0)),
            pl.BlockSpec((D, D), lambda i: (0, 0)),
            pl.BlockSpec((RB, NW), lambda i: (i, 0)),
        ],
        out_specs=[
            pl.BlockSpec((NC, RB, DH), lambda i: (0, i, 0)),
            pl.BlockSpec((RB, 1), lambda i: (i, 0)),
        ],
        out_shape=[
            jax.ShapeDtypeStruct((NC, N, DH), jnp.float32),
            jax.ShapeDtypeStruct((N, 1), jnp.float32),
        ],
    )(x, w1, hist)


def _mid(part, hs1, dinv, b1, w2):
    return pl.pallas_call(
        _mid_body,
        grid=(GRID,),
        in_specs=[
            pl.BlockSpec((NC, RB, DH), lambda i: (0, i, 0)),
            pl.BlockSpec((NC, RB, DH), lambda i: (0, i, 0)),
            pl.BlockSpec((RB, 1), lambda i: (i, 0)),
            pl.BlockSpec((1, D), lambda i: (0, 0)),
            pl.BlockSpec((D, D), lambda i: (0, 0)),
        ],
        out_specs=pl.BlockSpec((NC, RB, DH), lambda i: (0, i, 0)),
        out_shape=jax.ShapeDtypeStruct((NC, N, DH), jnp.float32),
    )(part, hs1, dinv, b1, w2)


def _final(part, hs2, dinv, b2):
    return pl.pallas_call(
        _out_body,
        grid=(GRID,),
        in_specs=[
            pl.BlockSpec((NC, RB, DH), lambda i: (0, i, 0)),
            pl.BlockSpec((NC, RB, DH), lambda i: (0, i, 0)),
            pl.BlockSpec((RB, 1), lambda i: (i, 0)),
            pl.BlockSpec((1, D), lambda i: (0, 0)),
        ],
        out_specs=pl.BlockSpec((RB, D), lambda i: (i, 0)),
        out_shape=jax.ShapeDtypeStruct((N, D), jnp.float32),
    )(part, hs2, dinv, b2)


@jax.jit
def kernel(x, edge_index, W1, b1, W2, b2):
    ei = edge_index.astype(jnp.int32)
    src = ei[0].reshape(NS, NCHUNK, CH)
    dst = ei[1].reshape(NS, NCHUNK, CH)
    dst_flat = ei[1].reshape(NW, EPW)
    b1r = b1.reshape(1, D)
    b2r = b2.reshape(1, D)

    hist = _degree_kernel(dst_flat)
    hs1, dinv = _mm1(x, W1, hist.T)
    part1 = _scatter_kernel(hs1, src, dst)
    hs2 = _mid(part1, hs1, dinv, b1r, W2)
    part2 = _scatter_kernel(hs2, src, dst)
    return _final(part2, hs2, dinv, b2r)
